# baseline (speedup 1.0000x reference)
import functools

import jax
import jax.numpy as jnp
from jax import lax
from jax.experimental import pallas as pl
from jax.experimental.pallas import tpu as pltpu
from jax.experimental.pallas import tpu_sc as plsc

B, T, D = 4, 4096, 2048
L = 16

_mesh = plsc.VectorSubcoreMesh(core_axis_name="c", subcore_axis_name="s")


@functools.partial(
    pl.kernel,
    mesh=_mesh,
    out_type=jax.ShapeDtypeStruct((B, D), jnp.float32),
    scratch_types=[
        pltpu.VMEM((L,), jnp.float32),
        pltpu.VMEM((L, D), jnp.float32),
        pltpu.SemaphoreType.DMA,
    ],
)
def _select_rows(table_hbm, scal_hbm, out_hbm, idx_v, rows_v, sem):
    nc = lax.axis_size("c")
    wid = lax.axis_index("s") * nc + lax.axis_index("c")

    @pl.when(wid == 0)
    def _():
        pltpu.sync_copy(scal_hbm, idx_v)
        v = idx_v[...].astype(jnp.int32)
        lane = lax.broadcasted_iota(jnp.int32, (L,), 0)
        flat = jnp.where(lane < B, v + lane * T, 0)
        pltpu.async_copy(table_hbm.at[flat], rows_v, sem).wait()
        pltpu.sync_copy(rows_v.at[pl.ds(0, B)], out_hbm)


def kernel(lstm_output, scalar_input):
    table = lstm_output.reshape(B * T, D)
    scal16 = jnp.zeros((L,), jnp.float32).at[:B].set(scalar_input[:, 0])
    return _select_rows(table, scal16)


# speedup vs baseline: 1.1000x; 1.1000x over previous
import functools

import jax
import jax.numpy as jnp
from jax import lax
from jax.experimental import pallas as pl
from jax.experimental.pallas import tpu as pltpu
from jax.experimental.pallas import tpu_sc as plsc

B, T, D = 4, 4096, 2048
L = 16

_mesh = plsc.VectorSubcoreMesh(core_axis_name="c", subcore_axis_name="s", num_cores=1)


@functools.partial(
    pl.kernel,
    mesh=_mesh,
    out_type=jax.ShapeDtypeStruct((B, D), jnp.float32),
    scratch_types=[
        pltpu.VMEM((L,), jnp.float32),
        pltpu.VMEM((L, D), jnp.float32),
        pltpu.SemaphoreType.DMA,
    ],
)
def _select_rows(table_hbm, scal_hbm, out_hbm, idx_v, rows_v, sem):
    nc = lax.axis_size("c")
    wid = lax.axis_index("s") * nc + lax.axis_index("c")

    @pl.when(wid == 0)
    def _():
        pltpu.sync_copy(scal_hbm, idx_v)
        v = idx_v[...].astype(jnp.int32)
        lane = lax.broadcasted_iota(jnp.int32, (L,), 0)
        flat = jnp.where(lane < B, v + lane * T, 0)
        pltpu.async_copy(table_hbm.at[flat], rows_v, sem).wait()
        pltpu.sync_copy(rows_v.at[pl.ds(0, B)], out_hbm)


def kernel(lstm_output, scalar_input):
    table = lstm_output.reshape(B * T, D)
    scal16 = jnp.zeros((L,), jnp.float32).at[:B].set(scalar_input[:, 0])
    return _select_rows(table, scal16)


# speedup vs baseline: 1.2703x; 1.1548x over previous
import functools

import jax
import jax.numpy as jnp
from jax.experimental import pallas as pl
from jax.experimental.pallas import tpu as pltpu
from jax.experimental.pallas import tpu_sc as plsc

B, T, D = 4, 4096, 2048

_mesh = plsc.ScalarSubcoreMesh(axis_name="c", num_cores=1)


@functools.partial(
    pl.kernel,
    mesh=_mesh,
    out_type=jax.ShapeDtypeStruct((B, D), jnp.float32),
    scratch_types=[
        pltpu.SMEM((B,), jnp.float32),
        pltpu.SemaphoreType.DMA,
    ],
)
def _select_rows(table_hbm, scal_hbm, out_hbm, idx_s, sem):
    pltpu.sync_copy(scal_hbm, idx_s)
    copies = []
    for b in range(B):
        r = idx_s[b].astype(jnp.int32) + b * T
        copies.append(
            pltpu.async_copy(table_hbm.at[pl.ds(r, 1)], out_hbm.at[pl.ds(b, 1)], sem)
        )
    for c in copies:
        c.wait()


def kernel(lstm_output, scalar_input):
    table = lstm_output.reshape(B * T, D)
    return _select_rows(table, scalar_input[:, 0])
